# baseline (speedup 1.0000x reference)
import functools

import jax
import jax.numpy as jnp
from jax import lax
from jax.experimental import pallas as pl
from jax.experimental.pallas import tpu as pltpu
from jax.experimental.pallas import tpu_sc as plsc

N = 10000
E = 320000
M = 2500
EQ = 80000
D = 128

NC = 2
NS = 16
NW = NC * NS

C1 = 128
EPT1 = 10240
EP1 = NW * EPT1
NCH1 = EPT1 // C1
FAST_C = 0
EPT1F = 16640
EPT1S = 3840
NCH1F = EPT1F // C1
NCH1S = EPT1S // C1
NP = 10240
ROWS1 = NP // NS

C2 = 128
EPT2 = 2560
EQP = NW * EPT2
NCH2 = EPT2 // C2
MP = 2560
ROWS2 = MP // NS
DW = 144

_mesh = plsc.VectorSubcoreMesh(core_axis_name="c", subcore_axis_name="s")


def _w_group(px_v, py_v, pz_v, qx_v, qy_v, qz_v, si, di):
    dx = plsc.load_gather(px_v, [si]) - plsc.load_gather(qx_v, [di])
    dy = plsc.load_gather(py_v, [si]) - plsc.load_gather(qy_v, [di])
    dz = plsc.load_gather(pz_v, [si]) - plsc.load_gather(qz_v, [di])
    return jnp.exp(-(dx * dx + dy * dy + dz * dz))


def _sc_w_body(px_hbm, py_hbm, pz_hbm, qx_hbm, qy_hbm, qz_hbm,
               src_hbm, dst_hbm, qs_hbm, qd_hbm,
               w1_hbm, w2_hbm,
               px_v, py_v, pz_v, qx_v, qy_v, qz_v,
               src_sl, dst_sl, w1_sl, qs_sl, qd_sl, w2_sl):
    c = lax.axis_index("c")
    s = lax.axis_index("s")
    wid = c * NS + s
    pltpu.sync_copy(px_hbm, px_v)
    pltpu.sync_copy(py_hbm, py_v)
    pltpu.sync_copy(pz_hbm, pz_v)
    pltpu.sync_copy(qx_hbm, qx_v)
    pltpu.sync_copy(qy_hbm, qy_v)
    pltpu.sync_copy(qz_hbm, qz_v)
    pltpu.sync_copy(src_hbm.at[pl.ds(wid * EPT1, EPT1)], src_sl)
    pltpu.sync_copy(dst_hbm.at[pl.ds(wid * EPT1, EPT1)], dst_sl)
    pltpu.sync_copy(qs_hbm.at[pl.ds(wid * EPT2, EPT2)], qs_sl)
    pltpu.sync_copy(qd_hbm.at[pl.ds(wid * EPT2, EPT2)], qd_sl)

    @plsc.parallel_loop(0, EPT1 // 16, unroll=4)
    def _(i):
        o = i * 16
        si = src_sl[pl.ds(o, 16)]
        di = dst_sl[pl.ds(o, 16)]
        w1_sl[pl.ds(o, 16)] = _w_group(px_v, py_v, pz_v,
                                       px_v, py_v, pz_v, si, di)

    pltpu.sync_copy(w1_sl, w1_hbm.at[pl.ds(wid * EPT1, EPT1)])

    @plsc.parallel_loop(0, EPT2 // 16, unroll=4)
    def _(i):
        o = i * 16
        si = qs_sl[pl.ds(o, 16)]
        di = qd_sl[pl.ds(o, 16)]
        w2_sl[pl.ds(o, 16)] = _w_group(px_v, py_v, pz_v,
                                       qx_v, qy_v, qz_v, si, di)

    pltpu.sync_copy(w2_sl, w2_hbm.at[pl.ds(wid * EPT2, EPT2)])


@functools.partial(
    pl.kernel,
    out_type=(jax.ShapeDtypeStruct((EP1,), jnp.float32),
              jax.ShapeDtypeStruct((EQP,), jnp.float32)),
    mesh=_mesh,
    scratch_types=[
        pltpu.VMEM((N,), jnp.float32),
        pltpu.VMEM((N,), jnp.float32),
        pltpu.VMEM((N,), jnp.float32),
        pltpu.VMEM((MP,), jnp.float32),
        pltpu.VMEM((MP,), jnp.float32),
        pltpu.VMEM((MP,), jnp.float32),
        pltpu.VMEM((EPT1,), jnp.int32),
        pltpu.VMEM((EPT1,), jnp.int32),
        pltpu.VMEM((EPT1,), jnp.float32),
        pltpu.VMEM((EPT2,), jnp.int32),
        pltpu.VMEM((EPT2,), jnp.int32),
        pltpu.VMEM((EPT2,), jnp.float32),
    ],
    compiler_params=pltpu.CompilerParams(needs_layout_passes=False),
)
def _sc_w(*refs):
    _sc_w_body(*refs)


def _scale_chunk(grows, srows, b, w_ref, w_off, nrows, extra_w):
    lane0 = jnp.arange(16, dtype=jnp.int32) == 0

    @plsc.parallel_loop(0, nrows, unroll=4)
    def _(r):
        wb = plsc.load_gather(w_ref, [jnp.full((16,), w_off + r,
                                               dtype=jnp.int32)])
        for j in range(D // 16):
            srows[b, r, pl.ds(j * 16, 16)] = grows[b, r, pl.ds(j * 16, 16)] * wb
        if extra_w:
            srows[b, r, pl.ds(D, 16)] = jnp.where(lane0, wb, 0.0)


def _sc_agg_body(src_hbm, dst_hbm, w_hbm, feat_hbm, z_hbm,
                 out_hbm,
                 src_b0, src_b1, dst_b0, dst_b1, w_b0, w_b1, grows,
                 gsem0, gsem1,
                 agg_sh):
    c = lax.axis_index("c")
    s = lax.axis_index("s")
    fast = c == FAST_C
    base0 = jnp.where(fast, s * EPT1F, NS * EPT1F + s * EPT1S)
    npairs = jnp.where(fast, NCH1F // 2, NCH1S // 2)
    src_bufs = (src_b0, src_b1)
    dst_bufs = (dst_b0, dst_b1)
    w_bufs = (w_b0, w_b1)
    gsems = (gsem0, gsem1)

    pltpu.sync_copy(z_hbm.at[pl.ds(s * ROWS1, ROWS1)],
                    agg_sh.at[pl.ds(s * ROWS1, ROWS1)])

    def load_src(g, b):
        pltpu.sync_copy(src_hbm.at[pl.ds(base0 + g * C1, C1)], src_bufs[b])

    def gather(b):
        pltpu.async_copy(feat_hbm.at[src_bufs[b]], grows.at[b], gsems[b])

    def gather_wait(b):
        pltpu.make_async_copy(feat_hbm.at[src_bufs[b]], grows.at[b],
                              gsems[b]).wait()

    load_src(0, 0)
    gather(0)
    load_src(1, 1)
    gather(1)
    plsc.subcore_barrier()

    def pair(g2, carry):
        for b in range(2):
            g = 2 * g2 + b
            pltpu.sync_copy(dst_hbm.at[pl.ds(base0 + g * C1, C1)],
                            dst_bufs[b])
            pltpu.sync_copy(w_hbm.at[pl.ds(base0 + g * C1, C1)],
                            w_bufs[b])
            gather_wait(b)

            @plsc.parallel_loop(0, C1, unroll=4)
            def _(r):
                wb = plsc.load_gather(
                    w_bufs[b], [jnp.full((16,), r, dtype=jnp.int32)])
                for j in range(D // 16):
                    grows[b, r, pl.ds(j * 16, 16)] = (
                        grows[b, r, pl.ds(j * 16, 16)] * wb)

            pltpu.sync_copy(grows.at[b], agg_sh.at[dst_bufs[b]], add=True)

            @pl.when(g2 < npairs - 1)
            def _():
                load_src(g + 2, b)
                gather(b)
        return carry

    lax.fori_loop(0, npairs, pair, 0)
    plsc.subcore_barrier()
    pltpu.sync_copy(agg_sh.at[pl.ds(s * ROWS1, ROWS1)],
                    out_hbm.at[c, pl.ds(s * ROWS1, ROWS1)])


@functools.partial(
    pl.kernel,
    out_type=jax.ShapeDtypeStruct((NC, NP, D), jnp.float32),
    mesh=_mesh,
    scratch_types=[
        pltpu.VMEM((C1,), jnp.int32),
        pltpu.VMEM((C1,), jnp.int32),
        pltpu.VMEM((C1,), jnp.int32),
        pltpu.VMEM((C1,), jnp.int32),
        pltpu.VMEM((C1,), jnp.float32),
        pltpu.VMEM((C1,), jnp.float32),
        pltpu.VMEM((2, C1, D), jnp.float32),
        pltpu.SemaphoreType.DMA,
        pltpu.SemaphoreType.DMA,
        pltpu.VMEM_SHARED((NP, D), jnp.float32),
    ],
    compiler_params=pltpu.CompilerParams(needs_layout_passes=False),
)
def _sc_agg(*refs):
    _sc_agg_body(*refs)


def _sc_pool_body(qs_hbm, qd_hbm, w_hbm, h_hbm, z_hbm,
                  out_hbm,
                  src_sl, dst_sl, w_sl, grows, srows, nd_sh,
                  gsem0, gsem1, ssem0, ssem1):
    c = lax.axis_index("c")
    s = lax.axis_index("s")
    wid = c * NS + s
    gsems = (gsem0, gsem1)
    ssems = (ssem0, ssem1)
    pltpu.sync_copy(qs_hbm.at[wid], src_sl)
    pltpu.sync_copy(qd_hbm.at[wid], dst_sl)
    pltpu.sync_copy(w_hbm.at[pl.ds(wid * EPT2, EPT2)], w_sl)
    pltpu.sync_copy(z_hbm.at[pl.ds(s * ROWS2, ROWS2)],
                    nd_sh.at[pl.ds(s * ROWS2, ROWS2)])
    plsc.subcore_barrier()

    def gather(g, b):
        pltpu.async_copy(h_hbm.at[src_sl.at[g]], grows.at[b], gsems[b])

    def gather_wait(g, b):
        pltpu.make_async_copy(h_hbm.at[src_sl.at[g]], grows.at[b],
                              gsems[b]).wait()

    def scatter(g, b):
        pltpu.async_copy(srows.at[b], nd_sh.at[dst_sl.at[g]], ssems[b],
                         add=True)

    def scatter_wait(g, b):
        pltpu.make_async_copy(srows.at[b], nd_sh.at[dst_sl.at[g]],
                              ssems[b]).wait()

    gather(0, 0)
    gather(1, 1)

    def pair(g2, carry):
        for b in range(2):
            g = 2 * g2 + b
            gather_wait(g, b)

            @pl.when(g2 >= 1)
            def _():
                scatter_wait(g - 2, b)

            _scale_chunk(grows, srows, b, w_sl, g * C2, C2, extra_w=True)

            @pl.when(g2 < NCH2 // 2 - 1)
            def _():
                gather(g + 2, b)

            scatter(g, b)
        return carry

    lax.fori_loop(0, NCH2 // 2, pair, 0)
    scatter_wait(NCH2 - 2, 0)
    scatter_wait(NCH2 - 1, 1)
    plsc.subcore_barrier()
    pltpu.sync_copy(nd_sh.at[pl.ds(s * ROWS2, ROWS2)],
                    out_hbm.at[c, pl.ds(s * ROWS2, ROWS2)])


@functools.partial(
    pl.kernel,
    out_type=jax.ShapeDtypeStruct((NC, MP, DW), jnp.float32),
    mesh=_mesh,
    scratch_types=[
        pltpu.VMEM((NCH2, C2), jnp.int32),
        pltpu.VMEM((NCH2, C2), jnp.int32),
        pltpu.VMEM((EPT2,), jnp.float32),
        pltpu.VMEM((2, C2, D), jnp.float32),
        pltpu.VMEM((2, C2, DW), jnp.float32),
        pltpu.VMEM_SHARED((MP, DW), jnp.float32),
        pltpu.SemaphoreType.DMA,
        pltpu.SemaphoreType.DMA,
        pltpu.SemaphoreType.DMA,
        pltpu.SemaphoreType.DMA,
    ],
    compiler_params=pltpu.CompilerParams(needs_layout_passes=False,
                                         use_tc_tiling_on_sc=False),
)
def _sc_pool(*refs):
    _sc_pool_body(*refs)


def _tc_h_body(feat_ref, a0_ref, a1_ref, w_ref, b_ref, out_ref):
    x = feat_ref[...] + a0_ref[0] + a1_ref[0]
    y = jnp.dot(x, w_ref[...], preferred_element_type=jnp.float32) + b_ref[...]
    out_ref[...] = y * jax.nn.sigmoid(y)


def _tc_h(feat, agg, w, b):
    bn = 1000
    grid = N // bn
    return pl.pallas_call(
        _tc_h_body,
        grid=(grid,),
        in_specs=[
            pl.BlockSpec((bn, D), lambda i: (i, 0)),
            pl.BlockSpec((1, bn, D), lambda i: (0, i, 0)),
            pl.BlockSpec((1, bn, D), lambda i: (1, i, 0)),
            pl.BlockSpec((D, D), lambda i: (0, 0)),
            pl.BlockSpec((1, D), lambda i: (0, 0)),
        ],
        out_specs=pl.BlockSpec((bn, D), lambda i: (i, 0)),
        out_shape=jax.ShapeDtypeStruct((N, D), jnp.float32),
    )(feat, agg, agg, w, b)


def _tc_head_body(nd_ref, wtf_ref, wwf_ref, gam_ref, bet_ref,
                  wl_ref, wb_ref, field_ref, wout_ref):
    nd = nd_ref[0] + nd_ref[1]
    num = nd[:, :D]
    den = nd[:, D:D + 1] + 1e-8
    pooled = num / den
    field_ref[...] = jnp.dot(pooled, wtf_ref[...],
                             preferred_element_type=jnp.float32)
    wf = jnp.dot(pooled, wwf_ref[...], preferred_element_type=jnp.float32)
    mu = jnp.mean(wf, axis=-1, keepdims=True)
    var = jnp.mean((wf - mu) ** 2, axis=-1, keepdims=True)
    ln = (wf - mu) * lax.rsqrt(var + 1e-5) * gam_ref[...] + bet_ref[...]
    sl = ln * jax.nn.sigmoid(ln)
    wout_ref[...] = jax.nn.sigmoid(
        jnp.dot(sl, wl_ref[...], preferred_element_type=jnp.float32)
        + wb_ref[...])


def _tc_head(nd, wtf, wwf, gam, bet, wl, wb):
    return pl.pallas_call(
        _tc_head_body,
        out_shape=(
            jax.ShapeDtypeStruct((MP, D), jnp.float32),
            jax.ShapeDtypeStruct((MP, 1), jnp.float32),
        ),
    )(nd, wtf, wwf, gam, bet, wl, wb)


def kernel(pos, feat, batch, edge_index, q_src, q_dst, W_extract, b_extract,
           W_tf, W_wf, ln_gamma, ln_beta, w_lin, w_bias):
    px = pos[:, 0]
    py = pos[:, 1]
    pz = pos[:, 2]
    src = edge_index[0]
    dst = edge_index[1]

    qpos = pos.reshape(M, N // M, 3)[:, 0, :]
    qx = jnp.pad(qpos[:, 0], (0, MP - M))
    qy = jnp.pad(qpos[:, 1], (0, MP - M))
    qz = jnp.pad(qpos[:, 2], (0, MP - M))
    srcp = jnp.pad(src, (0, EP1 - E))
    dstp = jnp.pad(dst, (0, EP1 - E), constant_values=NP - 1)
    qsp = jnp.pad(q_src, (0, EQP - EQ))
    qdp = jnp.pad(q_dst, (0, EQP - EQ), constant_values=MP - 1)

    w_edge, w_query = _sc_w(px, py, pz, qx, qy, qz, srcp, dstp, qsp, qdp)
    agg = _sc_agg(srcp, dstp, w_edge, feat,
                  jnp.zeros((NP, D), dtype=jnp.float32))

    h = _tc_h(feat, agg, W_extract, b_extract.reshape(1, D))

    qs_rs = qsp.reshape(NW, NCH2, C2)
    qd_rs = qdp.reshape(NW, NCH2, C2)

    nd = _sc_pool(qs_rs, qd_rs, w_query, h,
                  jnp.zeros((MP, DW), dtype=jnp.float32))

    field_p, w_p = _tc_head(nd, W_tf, W_wf,
                            ln_gamma.reshape(1, D), ln_beta.reshape(1, D),
                            w_lin, w_bias.reshape(1, 1))
    return qpos, field_p[:M], w_p[:M, 0]


# speedup vs baseline: 1.0428x; 1.0428x over previous
import functools

import jax
import jax.numpy as jnp
from jax import lax
from jax.experimental import pallas as pl
from jax.experimental.pallas import tpu as pltpu
from jax.experimental.pallas import tpu_sc as plsc

N = 10000
E = 320000
M = 2500
EQ = 80000
D = 128

NC = 2
NS = 16
NW = NC * NS

C1 = 128
EPT1 = 10240
EP1 = NW * EPT1
NCH1 = EPT1 // C1
FAST_C = 0
EPT1F = 13568
EPT1S = 6912
NCH1F = EPT1F // C1
NCH1S = EPT1S // C1
NP = 10240
ROWS1 = NP // NS

C2 = 128
EPT2 = 2560
EQP = NW * EPT2
NCH2 = EPT2 // C2
EPT2F = 3328
EPT2S = 1792
NCH2F = EPT2F // C2
NCH2S = EPT2S // C2
MP = 2560
ROWS2 = MP // NS
DW = 144

_mesh = plsc.VectorSubcoreMesh(core_axis_name="c", subcore_axis_name="s")


def _w_group(px_v, py_v, pz_v, qx_v, qy_v, qz_v, si, di):
    dx = plsc.load_gather(px_v, [si]) - plsc.load_gather(qx_v, [di])
    dy = plsc.load_gather(py_v, [si]) - plsc.load_gather(qy_v, [di])
    dz = plsc.load_gather(pz_v, [si]) - plsc.load_gather(qz_v, [di])
    return jnp.exp(-(dx * dx + dy * dy + dz * dz))


def _sc_w_body(px_hbm, py_hbm, pz_hbm, qx_hbm, qy_hbm, qz_hbm,
               src_hbm, dst_hbm, qs_hbm, qd_hbm,
               w1_hbm, w2_hbm,
               px_v, py_v, pz_v, qx_v, qy_v, qz_v,
               src_sl, dst_sl, w1_sl, qs_sl, qd_sl, w2_sl):
    c = lax.axis_index("c")
    s = lax.axis_index("s")
    wid = c * NS + s
    pltpu.sync_copy(px_hbm, px_v)
    pltpu.sync_copy(py_hbm, py_v)
    pltpu.sync_copy(pz_hbm, pz_v)
    pltpu.sync_copy(qx_hbm, qx_v)
    pltpu.sync_copy(qy_hbm, qy_v)
    pltpu.sync_copy(qz_hbm, qz_v)
    pltpu.sync_copy(src_hbm.at[pl.ds(wid * EPT1, EPT1)], src_sl)
    pltpu.sync_copy(dst_hbm.at[pl.ds(wid * EPT1, EPT1)], dst_sl)
    pltpu.sync_copy(qs_hbm.at[pl.ds(wid * EPT2, EPT2)], qs_sl)
    pltpu.sync_copy(qd_hbm.at[pl.ds(wid * EPT2, EPT2)], qd_sl)

    @plsc.parallel_loop(0, EPT1 // 16, unroll=4)
    def _(i):
        o = i * 16
        si = src_sl[pl.ds(o, 16)]
        di = dst_sl[pl.ds(o, 16)]
        w1_sl[pl.ds(o, 16)] = _w_group(px_v, py_v, pz_v,
                                       px_v, py_v, pz_v, si, di)

    pltpu.sync_copy(w1_sl, w1_hbm.at[pl.ds(wid * EPT1, EPT1)])

    @plsc.parallel_loop(0, EPT2 // 16, unroll=4)
    def _(i):
        o = i * 16
        si = qs_sl[pl.ds(o, 16)]
        di = qd_sl[pl.ds(o, 16)]
        w2_sl[pl.ds(o, 16)] = _w_group(px_v, py_v, pz_v,
                                       qx_v, qy_v, qz_v, si, di)

    pltpu.sync_copy(w2_sl, w2_hbm.at[pl.ds(wid * EPT2, EPT2)])


@functools.partial(
    pl.kernel,
    out_type=(jax.ShapeDtypeStruct((EP1,), jnp.float32),
              jax.ShapeDtypeStruct((EQP,), jnp.float32)),
    mesh=_mesh,
    scratch_types=[
        pltpu.VMEM((N,), jnp.float32),
        pltpu.VMEM((N,), jnp.float32),
        pltpu.VMEM((N,), jnp.float32),
        pltpu.VMEM((MP,), jnp.float32),
        pltpu.VMEM((MP,), jnp.float32),
        pltpu.VMEM((MP,), jnp.float32),
        pltpu.VMEM((EPT1,), jnp.int32),
        pltpu.VMEM((EPT1,), jnp.int32),
        pltpu.VMEM((EPT1,), jnp.float32),
        pltpu.VMEM((EPT2,), jnp.int32),
        pltpu.VMEM((EPT2,), jnp.int32),
        pltpu.VMEM((EPT2,), jnp.float32),
    ],
    compiler_params=pltpu.CompilerParams(needs_layout_passes=False),
)
def _sc_w(*refs):
    _sc_w_body(*refs)


def _scale_chunk(grows, srows, b, w_ref, w_off, nrows, extra_w):
    lane0 = jnp.arange(16, dtype=jnp.int32) == 0

    @plsc.parallel_loop(0, nrows, unroll=4)
    def _(r):
        wb = plsc.load_gather(w_ref, [jnp.full((16,), w_off + r,
                                               dtype=jnp.int32)])
        for j in range(D // 16):
            srows[b, r, pl.ds(j * 16, 16)] = grows[b, r, pl.ds(j * 16, 16)] * wb
        if extra_w:
            srows[b, r, pl.ds(D, 16)] = jnp.where(lane0, wb, 0.0)


def _zero_shared(zbuf, acc_sh, s, rows_per_tile, buf_rows, width):
    zero = jnp.zeros((16,), dtype=jnp.float32)

    def zbody(r, carry):
        for j in range(width // 16):
            zbuf[r, pl.ds(j * 16, 16)] = zero
        return carry

    lax.fori_loop(0, buf_rows, zbody, 0)
    base = s * rows_per_tile
    full, rem = divmod(rows_per_tile, buf_rows)
    for t in range(full):
        pltpu.sync_copy(zbuf,
                        acc_sh.at[pl.ds(base + t * buf_rows, buf_rows)])
    if rem:
        pltpu.sync_copy(zbuf.at[pl.ds(0, rem)],
                        acc_sh.at[pl.ds(base + full * buf_rows, rem)])


def _sc_agg_body(src_hbm, dst_hbm, w_hbm, feat_hbm,
                 out_hbm,
                 src_b0, src_b1, dst_b0, dst_b1, w_sl, grows,
                 gsem0, gsem1,
                 agg_sh):
    c = lax.axis_index("c")
    s = lax.axis_index("s")
    fast = c == FAST_C
    base0 = jnp.where(fast, s * EPT1F, NS * EPT1F + s * EPT1S)
    npairs = jnp.where(fast, NCH1F // 2, NCH1S // 2)
    src_bufs = (src_b0, src_b1)
    dst_bufs = (dst_b0, dst_b1)
    gsems = (gsem0, gsem1)

    @pl.when(fast)
    def _():
        pltpu.sync_copy(w_hbm.at[pl.ds(base0, EPT1F)], w_sl)

    @pl.when(jnp.logical_not(fast))
    def _():
        pltpu.sync_copy(w_hbm.at[pl.ds(base0, EPT1S)],
                        w_sl.at[pl.ds(0, EPT1S)])

    _zero_shared(grows.at[0], agg_sh, s, ROWS1, C1, D)

    def load_src(g, b):
        pltpu.sync_copy(src_hbm.at[pl.ds(base0 + g * C1, C1)], src_bufs[b])

    def gather(b):
        pltpu.async_copy(feat_hbm.at[src_bufs[b]], grows.at[b], gsems[b])

    def gather_wait(b):
        pltpu.make_async_copy(feat_hbm.at[src_bufs[b]], grows.at[b],
                              gsems[b]).wait()

    load_src(0, 0)
    gather(0)
    load_src(1, 1)
    gather(1)
    plsc.subcore_barrier()

    def pair(g2, carry):
        for b in range(2):
            g = 2 * g2 + b
            pltpu.sync_copy(dst_hbm.at[pl.ds(base0 + g * C1, C1)],
                            dst_bufs[b])
            gather_wait(b)

            @plsc.parallel_loop(0, C1, unroll=4)
            def _(r):
                wb = plsc.load_gather(
                    w_sl, [jnp.full((16,), g * C1 + r, dtype=jnp.int32)])
                for j in range(D // 16):
                    grows[b, r, pl.ds(j * 16, 16)] = (
                        grows[b, r, pl.ds(j * 16, 16)] * wb)

            pltpu.sync_copy(grows.at[b], agg_sh.at[dst_bufs[b]], add=True)

            @pl.when(g2 < npairs - 1)
            def _():
                load_src(g + 2, b)
                gather(b)
        return carry

    lax.fori_loop(0, npairs, pair, 0)
    plsc.subcore_barrier()
    pltpu.sync_copy(agg_sh.at[pl.ds(s * ROWS1, ROWS1)],
                    out_hbm.at[c, pl.ds(s * ROWS1, ROWS1)])


@functools.partial(
    pl.kernel,
    out_type=jax.ShapeDtypeStruct((NC, NP, D), jnp.float32),
    mesh=_mesh,
    scratch_types=[
        pltpu.VMEM((C1,), jnp.int32),
        pltpu.VMEM((C1,), jnp.int32),
        pltpu.VMEM((C1,), jnp.int32),
        pltpu.VMEM((C1,), jnp.int32),
        pltpu.VMEM((EPT1F,), jnp.float32),
        pltpu.VMEM((2, C1, D), jnp.float32),
        pltpu.SemaphoreType.DMA,
        pltpu.SemaphoreType.DMA,
        pltpu.VMEM_SHARED((NP, D), jnp.float32),
    ],
    compiler_params=pltpu.CompilerParams(needs_layout_passes=False),
)
def _sc_agg(*refs):
    _sc_agg_body(*refs)


def _sc_pool_body(qs_f_hbm, qs_s_hbm, qd_f_hbm, qd_s_hbm, w_hbm, h_hbm,
                  out_hbm,
                  src_sl, dst_sl, w_sl, grows, srows, nd_sh,
                  gsem0, gsem1, ssem0, ssem1):
    c = lax.axis_index("c")
    s = lax.axis_index("s")
    fast = c == FAST_C
    base0 = jnp.where(fast, s * EPT2F, NS * EPT2F + s * EPT2S)
    npairs = jnp.where(fast, NCH2F // 2, NCH2S // 2)
    gsems = (gsem0, gsem1)
    ssems = (ssem0, ssem1)

    @pl.when(fast)
    def _():
        pltpu.sync_copy(qs_f_hbm.at[s], src_sl)
        pltpu.sync_copy(qd_f_hbm.at[s], dst_sl)
        pltpu.sync_copy(w_hbm.at[pl.ds(base0, EPT2F)], w_sl)

    @pl.when(jnp.logical_not(fast))
    def _():
        pltpu.sync_copy(qs_s_hbm.at[s], src_sl.at[pl.ds(0, NCH2S)])
        pltpu.sync_copy(qd_s_hbm.at[s], dst_sl.at[pl.ds(0, NCH2S)])
        pltpu.sync_copy(w_hbm.at[pl.ds(base0, EPT2S)],
                        w_sl.at[pl.ds(0, EPT2S)])

    _zero_shared(srows.at[0], nd_sh, s, ROWS2, C2, DW)
    plsc.subcore_barrier()

    def gather(g, b):
        pltpu.async_copy(h_hbm.at[src_sl.at[g]], grows.at[b], gsems[b])

    def gather_wait(g, b):
        pltpu.make_async_copy(h_hbm.at[src_sl.at[g]], grows.at[b],
                              gsems[b]).wait()

    def scatter(g, b):
        pltpu.async_copy(srows.at[b], nd_sh.at[dst_sl.at[g]], ssems[b],
                         add=True)

    def scatter_wait(g, b):
        pltpu.make_async_copy(srows.at[b], nd_sh.at[dst_sl.at[g]],
                              ssems[b]).wait()

    gather(0, 0)
    gather(1, 1)

    def pair(g2, carry):
        for b in range(2):
            g = 2 * g2 + b
            gather_wait(g, b)

            @pl.when(g2 >= 1)
            def _():
                scatter_wait(g - 2, b)

            _scale_chunk(grows, srows, b, w_sl, g * C2, C2, extra_w=True)

            @pl.when(g2 < npairs - 1)
            def _():
                gather(g + 2, b)

            scatter(g, b)
        return carry

    lax.fori_loop(0, npairs, pair, 0)
    scatter_wait(2 * npairs - 2, 0)
    scatter_wait(2 * npairs - 1, 1)
    plsc.subcore_barrier()
    pltpu.sync_copy(nd_sh.at[pl.ds(s * ROWS2, ROWS2)],
                    out_hbm.at[c, pl.ds(s * ROWS2, ROWS2)])


@functools.partial(
    pl.kernel,
    out_type=jax.ShapeDtypeStruct((NC, MP, DW), jnp.float32),
    mesh=_mesh,
    scratch_types=[
        pltpu.VMEM((NCH2F, C2), jnp.int32),
        pltpu.VMEM((NCH2F, C2), jnp.int32),
        pltpu.VMEM((EPT2F,), jnp.float32),
        pltpu.VMEM((2, C2, D), jnp.float32),
        pltpu.VMEM((2, C2, DW), jnp.float32),
        pltpu.VMEM_SHARED((MP, DW), jnp.float32),
        pltpu.SemaphoreType.DMA,
        pltpu.SemaphoreType.DMA,
        pltpu.SemaphoreType.DMA,
        pltpu.SemaphoreType.DMA,
    ],
    compiler_params=pltpu.CompilerParams(needs_layout_passes=False,
                                         use_tc_tiling_on_sc=False),
)
def _sc_pool(*refs):
    _sc_pool_body(*refs)


def _tc_h_body(feat_ref, a0_ref, a1_ref, w_ref, b_ref, out_ref):
    x = feat_ref[...] + a0_ref[0] + a1_ref[0]
    y = jnp.dot(x, w_ref[...], preferred_element_type=jnp.float32) + b_ref[...]
    out_ref[...] = y * jax.nn.sigmoid(y)


def _tc_h(feat, agg, w, b):
    bn = 1000
    grid = N // bn
    return pl.pallas_call(
        _tc_h_body,
        grid=(grid,),
        in_specs=[
            pl.BlockSpec((bn, D), lambda i: (i, 0)),
            pl.BlockSpec((1, bn, D), lambda i: (0, i, 0)),
            pl.BlockSpec((1, bn, D), lambda i: (1, i, 0)),
            pl.BlockSpec((D, D), lambda i: (0, 0)),
            pl.BlockSpec((1, D), lambda i: (0, 0)),
        ],
        out_specs=pl.BlockSpec((bn, D), lambda i: (i, 0)),
        out_shape=jax.ShapeDtypeStruct((N, D), jnp.float32),
    )(feat, agg, agg, w, b)


def _tc_head_body(nd_ref, wtf_ref, wwf_ref, gam_ref, bet_ref,
                  wl_ref, wb_ref, field_ref, wout_ref):
    nd = nd_ref[0] + nd_ref[1]
    num = nd[:, :D]
    den = nd[:, D:D + 1] + 1e-8
    pooled = num / den
    field_ref[...] = jnp.dot(pooled, wtf_ref[...],
                             preferred_element_type=jnp.float32)
    wf = jnp.dot(pooled, wwf_ref[...], preferred_element_type=jnp.float32)
    mu = jnp.mean(wf, axis=-1, keepdims=True)
    var = jnp.mean((wf - mu) ** 2, axis=-1, keepdims=True)
    ln = (wf - mu) * lax.rsqrt(var + 1e-5) * gam_ref[...] + bet_ref[...]
    sl = ln * jax.nn.sigmoid(ln)
    wout_ref[...] = jax.nn.sigmoid(
        jnp.dot(sl, wl_ref[...], preferred_element_type=jnp.float32)
        + wb_ref[...])


def _tc_head(nd, wtf, wwf, gam, bet, wl, wb):
    return pl.pallas_call(
        _tc_head_body,
        out_shape=(
            jax.ShapeDtypeStruct((MP, D), jnp.float32),
            jax.ShapeDtypeStruct((MP, 1), jnp.float32),
        ),
    )(nd, wtf, wwf, gam, bet, wl, wb)


def kernel(pos, feat, batch, edge_index, q_src, q_dst, W_extract, b_extract,
           W_tf, W_wf, ln_gamma, ln_beta, w_lin, w_bias):
    px = pos[:, 0]
    py = pos[:, 1]
    pz = pos[:, 2]
    src = edge_index[0]
    dst = edge_index[1]

    qpos = pos.reshape(M, N // M, 3)[:, 0, :]
    qx = jnp.pad(qpos[:, 0], (0, MP - M))
    qy = jnp.pad(qpos[:, 1], (0, MP - M))
    qz = jnp.pad(qpos[:, 2], (0, MP - M))
    srcp = jnp.pad(src, (0, EP1 - E))
    dstp = jnp.pad(dst, (0, EP1 - E), constant_values=NP - 1)
    qsp = jnp.pad(q_src, (0, EQP - EQ))
    qdp = jnp.pad(q_dst, (0, EQP - EQ), constant_values=MP - 1)

    w_edge, w_query = _sc_w(px, py, pz, qx, qy, qz, srcp, dstp, qsp, qdp)
    agg = _sc_agg(srcp, dstp, w_edge, feat)

    h = _tc_h(feat, agg, W_extract, b_extract.reshape(1, D))

    cut = NS * EPT2F
    qs_f = qsp[:cut].reshape(NS, NCH2F, C2)
    qs_s = qsp[cut:].reshape(NS, NCH2S, C2)
    qd_f = qdp[:cut].reshape(NS, NCH2F, C2)
    qd_s = qdp[cut:].reshape(NS, NCH2S, C2)

    nd = _sc_pool(qs_f, qs_s, qd_f, qd_s, w_query, h)

    field_p, w_p = _tc_head(nd, W_tf, W_wf,
                            ln_gamma.reshape(1, D), ln_beta.reshape(1, D),
                            w_lin, w_bias.reshape(1, 1))
    return qpos, field_p[:M], w_p[:M, 0]
